# baseline (speedup 1.0000x reference)
import functools

import jax
import jax.numpy as jnp
from jax import lax
from jax.experimental import pallas as pl
from jax.experimental.pallas import tpu as pltpu
from jax.experimental.pallas import tpu_sc as plsc

B = 16
S = 4096
D = 1024
S_TC = 2048
DC = 256
NCH = D // DC
NSEG = 3 * B
NTASK = NSEG * NCH
NTILE = 32
SLOTS = NTASK // NTILE
BLKR = 128
LOG2_BLKR = 7
NLANE = 16
NVEC = DC // NLANE
BS_TC = 512


def _scal_i(ref, slot, f):
    return ref[slot, f, :][0]


def _scal_f(ref, slot, f):
    return ref[slot, f, :][0]


def _sc_body(x_hbm, mi_hbm, mf_hbm, out_hbm, miv, mfv, bufA, bufB, clsbuf,
             outv, semA, semB, semC, semD):
    wid = lax.axis_index("c") * 16 + lax.axis_index("s")
    pltpu.sync_copy(mi_hbm.at[wid], miv)
    pltpu.sync_copy(mf_hbm.at[wid], mfv)

    tasks = []
    for slot in range(SLOTS):
        b = _scal_i(miv, slot, 0)
        s0 = _scal_i(miv, slot, 1)
        nr = _scal_i(miv, slot, 2)
        dc = pl.multiple_of(_scal_i(miv, slot, 3), DC)
        seg = _scal_i(miv, slot, 4)
        cr = _scal_i(miv, slot, 5)
        inv = _scal_f(mfv, slot, 0)
        fl = _scal_f(mfv, slot, 1)
        a8 = jnp.bitwise_and(s0, jnp.int32(~7))
        nb = lax.shift_right_logical(s0 - a8 + nr + (BLKR - 1), LOG2_BLKR)
        nb = jnp.where(nr > 0, nb, 0)
        cra = pl.multiple_of(jnp.bitwise_and(cr, jnp.int32(~7)), 8)
        tasks.append((b, s0, nr, dc, seg, cr, inv, fl, a8, nb, cra))

    def blk_copy(t, k, buf, sem):
        b, _, _, dc, _, _, _, _, a8, _, _ = tasks[t]
        sblk = a8 + k * BLKR
        sc = pl.multiple_of(jnp.minimum(sblk, S - BLKR), 8)
        return pltpu.make_async_copy(
            x_hbm.at[b, pl.ds(sc, BLKR), pl.ds(dc, DC)], buf, sem)

    def cls_copy(t):
        b, _, _, dc, _, _, _, _, _, _, cra = tasks[t]
        return pltpu.make_async_copy(
            x_hbm.at[b, pl.ds(cra, 8), pl.ds(dc, DC)], clsbuf, semC)

    def prime(t):
        @pl.when(tasks[t][9] > 0)
        def _():
            blk_copy(t, 0, bufA, semA).start()

    prime(0)
    cls_copy(0).start()
    prev_out_cp = None

    for slot in range(SLOTS):
        b, s0, nr, dc, seg, cr, inv, fl, a8, nb, cra = tasks[slot]

        def accum(k, acc, buf, s0=s0, nr=nr, a8=a8):
            sblk = a8 + k * BLKR
            sc = jnp.minimum(sblk, S - BLKR)
            lo = jnp.maximum(s0, sblk) - sc
            hi = jnp.minimum(s0 + nr, sblk + BLKR) - sc

            def rbody(r, a):
                return tuple(a[c] + buf[r, pl.ds(c * NLANE, NLANE)]
                             for c in range(NVEC))

            return lax.fori_loop(lo, hi, rbody, acc)

        def bbody(j, acc, slot=slot, nb=nb):
            k0 = 2 * j

            @pl.when(k0 + 1 < nb)
            def _():
                blk_copy(slot, k0 + 1, bufB, semB).start()

            blk_copy(slot, k0, bufA, semA).wait()
            acc = accum(k0, acc, bufA)

            @pl.when(k0 + 2 < nb)
            def _():
                blk_copy(slot, k0 + 2, bufA, semA).start()

            @pl.when(k0 + 1 < nb)
            def _():
                blk_copy(slot, k0 + 1, bufB, semB).wait()

            return accum(k0 + 1, acc, bufB)

        acc0 = tuple(jnp.zeros((NLANE,), jnp.float32) for _ in range(NVEC))
        nj = lax.shift_right_logical(nb + 1, 1)
        acc = lax.fori_loop(0, nj, bbody, acc0)

        if slot + 1 < SLOTS:
            prime(slot + 1)

        cls_copy(slot).wait()
        if prev_out_cp is not None:
            prev_out_cp.wait()
        coff = cr - cra
        for c in range(NVEC):
            outv[pl.ds(c * NLANE, NLANE)] = (
                acc[c] + fl * clsbuf[coff, pl.ds(c * NLANE, NLANE)]) * inv
        prev_out_cp = pltpu.make_async_copy(
            outv, out_hbm.at[pl.ds(pl.multiple_of(seg * D + dc, DC), DC)],
            semD)
        prev_out_cp.start()
        if slot + 1 < SLOTS:
            cls_copy(slot + 1).start()

    prev_out_cp.wait()


def _build_meta(h0, h1, r0, r1, t0, t1):
    barr = jnp.arange(B, dtype=jnp.int32)
    s0f = jnp.concatenate([h0, r0, t0])
    s1 = jnp.concatenate([h1, r1, t1])
    wf = s1 - s0f + 1
    s0 = jnp.maximum(s0f, S_TC)
    w = jnp.maximum(s1 - s0 + 1, 0)
    flag = jnp.concatenate([jnp.zeros(B), jnp.ones(B), jnp.ones(B)]
                           ).astype(jnp.float32)
    clsrow = jnp.concatenate([h0, h0, h0])
    flag = flag * (clsrow >= S_TC).astype(jnp.float32)
    segb = jnp.concatenate([barr, barr, barr])
    inv = 1.0 / (wf.astype(jnp.float32) +
                 jnp.concatenate([jnp.zeros(B), jnp.ones(B), jnp.ones(B)]))
    segid = jnp.arange(NSEG, dtype=jnp.int32)

    tseg = jnp.repeat(segid, NCH)
    tdc = jnp.tile(jnp.arange(NCH, dtype=jnp.int32) * DC, NSEG)
    tb = jnp.repeat(segb, NCH)
    ts0 = jnp.repeat(s0, NCH)
    tw = jnp.repeat(w, NCH)
    tcls = jnp.repeat(clsrow, NCH)
    tflag = jnp.repeat(flag, NCH)
    tinv = jnp.repeat(inv, NCH)

    fi = jnp.stack([tb, ts0, tw, tdc, tseg, tcls,
                    jnp.zeros_like(tb), jnp.zeros_like(tb)],
                   axis=-1).astype(jnp.int32)
    ff = jnp.stack([tinv, tflag], axis=-1).astype(jnp.float32)

    perm = jnp.argsort(-tw)
    cols = jnp.arange(NTILE).reshape(2, NTILE // 2).T.ravel()
    serp = jnp.stack([cols if r % 2 == 0 else cols[::-1]
                      for r in range(SLOTS)])
    pos = jnp.argsort(serp, axis=1) + (jnp.arange(SLOTS)[:, None] * NTILE)
    assign = perm[pos].T

    mi = jnp.broadcast_to(fi[assign][..., None],
                          (NTILE, SLOTS, 8, NLANE)).astype(jnp.int32)
    mf = jnp.broadcast_to(ff[assign][..., None],
                          (NTILE, SLOTS, 2, NLANE)).astype(jnp.float32)
    return mi, mf


@functools.cache
def _pool_kernel():
    return functools.partial(
        pl.kernel,
        out_type=jax.ShapeDtypeStruct((NSEG * D,), jnp.float32),
        mesh=plsc.VectorSubcoreMesh(core_axis_name="c", subcore_axis_name="s"),
        scratch_types=[
            pltpu.VMEM((SLOTS, 8, NLANE), jnp.int32),
            pltpu.VMEM((SLOTS, 2, NLANE), jnp.float32),
            pltpu.VMEM((BLKR, DC), jnp.float32),
            pltpu.VMEM((BLKR, DC), jnp.float32),
            pltpu.VMEM((8, DC), jnp.float32),
            pltpu.VMEM((DC,), jnp.float32),
            pltpu.SemaphoreType.DMA,
            pltpu.SemaphoreType.DMA,
            pltpu.SemaphoreType.DMA,
            pltpu.SemaphoreType.DMA,
        ],
    )(_sc_body)


def _tc_body(spans_ref, blk_ref, valid_ref, x_ref, out_ref):
    s = pl.program_id(1)
    b = pl.program_id(0)
    h0 = spans_ref[b, 0]
    h1 = spans_ref[b, 1]
    r0 = spans_ref[b, 2]
    r1 = spans_ref[b, 3]
    t0 = spans_ref[b, 4]
    t1 = spans_ref[b, 5]

    sblk = blk_ref[b, s]
    validf = valid_ref[b, s].astype(jnp.float32)
    pos = jax.lax.broadcasted_iota(jnp.int32, (1, BS_TC), 1) + sblk * BS_TC
    onehot = (pos == h0).astype(jnp.float32)

    def span_w(lo, hi, with_cls):
        m = ((pos >= lo) & (pos <= hi)).astype(jnp.float32)
        return m + onehot if with_cls else m

    wts = validf * jnp.concatenate([
        span_w(h0, h1, False),
        span_w(r0, r1, True),
        span_w(t0, t1, True),
    ], axis=0)

    contrib = jnp.dot(wts, x_ref[0], preferred_element_type=jnp.float32)

    @pl.when(s == 0)
    def _():
        out_ref[...] = jnp.zeros_like(out_ref)

    out_ref[0] += contrib

    @pl.when(s == (S_TC // BS_TC) - 1)
    def _():
        inv_h = 1.0 / (h1 - h0 + 1).astype(jnp.float32)
        inv_r = 1.0 / (r1 - r0 + 2).astype(jnp.float32)
        inv_t = 1.0 / (t1 - t0 + 2).astype(jnp.float32)
        out_ref[0, 0:1, :] *= inv_h
        out_ref[0, 1:2, :] *= inv_r
        out_ref[0, 2:3, :] *= inv_t


def _build_blockmap(spans):
    nblk = S_TC // BS_TC
    blk = jnp.arange(nblk, dtype=jnp.int32)
    lo = spans[:, 0::2].astype(jnp.int32)
    hi = jnp.minimum(spans[:, 1::2], S_TC - 1).astype(jnp.int32)
    touched = ((blk[None, :, None] * BS_TC <= hi[:, None, :]) &
               ((blk[None, :, None] + 1) * BS_TC > lo[:, None, :]) &
               (lo <= hi)[:, None, :]).any(-1)
    key = jnp.where(touched, blk[None, :], nblk)
    srt = jnp.sort(key, axis=1)
    cnt = touched.sum(axis=1, keepdims=True)
    pos = jnp.minimum(blk[None, :], jnp.maximum(cnt - 1, 0))
    blkmap = jnp.take_along_axis(srt, pos, axis=1)
    blkmap = jnp.where(blkmap == nblk, 0, blkmap).astype(jnp.int32)
    valid = (blk[None, :] < cnt).astype(jnp.int32)
    return blkmap, valid


@functools.cache
def _tc_kernel():
    return pl.pallas_call(
        _tc_body,
        grid_spec=pltpu.PrefetchScalarGridSpec(
            num_scalar_prefetch=3,
            grid=(B, S_TC // BS_TC),
            in_specs=[pl.BlockSpec((1, BS_TC, D),
                                   lambda b, s, spans, blk, valid:
                                   (b, blk[b, s], 0))],
            out_specs=pl.BlockSpec((1, 3, D),
                                   lambda b, s, *_: (b, 0, 0)),
        ),
        out_shape=jax.ShapeDtypeStruct((B, 3, D), jnp.float32),
        compiler_params=pltpu.CompilerParams(
            dimension_semantics=("parallel", "arbitrary")),
    )


def kernel(last_hidden_state, head_text_idx, relation_text_idx, tail_text_idx):
    hidx = head_text_idx.astype(jnp.int32)
    ridx = relation_text_idx.astype(jnp.int32)
    tidx = tail_text_idx.astype(jnp.int32)
    mi, mf = _build_meta(hidx[:, 0], hidx[:, 1], ridx[:, 0], ridx[:, 1],
                         tidx[:, 0], tidx[:, 1])
    sc_out = _pool_kernel()(last_hidden_state, mi, mf).reshape(NSEG, D)

    spans = jnp.concatenate([hidx, ridx, tidx], axis=-1)
    spans = jnp.pad(spans, ((0, 0), (0, 2)))
    blkmap, valid = _build_blockmap(spans)
    tc_out = _tc_kernel()(spans, blkmap, valid, last_hidden_state)

    head = tc_out[:, 0, :] + sc_out[:B]
    rel = tc_out[:, 1, :] + sc_out[B:2 * B]
    tail = tc_out[:, 2, :] + sc_out[2 * B:]
    return (head, rel, tail)


# speedup vs baseline: 1.0458x; 1.0458x over previous
import functools

import jax
import jax.numpy as jnp
from jax import lax
from jax.experimental import pallas as pl
from jax.experimental.pallas import tpu as pltpu
from jax.experimental.pallas import tpu_sc as plsc

B = 16
S = 4096
D = 1024
S_TC = 2048
DC = 256
NCH = D // DC
NSEG = 3 * B
NTASK = NSEG * NCH
NTILE = 32
SLOTS = NTASK // NTILE
BLKR = 128
LOG2_BLKR = 7
NLANE = 16
NVEC = DC // NLANE
BS_TC = 1024


def _scal_i(ref, slot, f):
    return ref[slot, f, :][0]


def _scal_f(ref, slot, f):
    return ref[slot, f, :][0]


def _sc_body(x_hbm, mi_hbm, mf_hbm, out_hbm, miv, mfv, bufA, bufB, clsbuf,
             outv, semA, semB, semC, semD):
    wid = lax.axis_index("c") * 16 + lax.axis_index("s")
    pltpu.sync_copy(mi_hbm.at[wid], miv)
    pltpu.sync_copy(mf_hbm.at[wid], mfv)

    tasks = []
    for slot in range(SLOTS):
        b = _scal_i(miv, slot, 0)
        s0 = _scal_i(miv, slot, 1)
        nr = _scal_i(miv, slot, 2)
        dc = pl.multiple_of(_scal_i(miv, slot, 3), DC)
        seg = _scal_i(miv, slot, 4)
        cr = _scal_i(miv, slot, 5)
        inv = _scal_f(mfv, slot, 0)
        fl = _scal_f(mfv, slot, 1)
        a8 = jnp.bitwise_and(s0, jnp.int32(~7))
        nb = lax.shift_right_logical(s0 - a8 + nr + (BLKR - 1), LOG2_BLKR)
        nb = jnp.where(nr > 0, nb, 0)
        cra = pl.multiple_of(jnp.bitwise_and(cr, jnp.int32(~7)), 8)
        tasks.append((b, s0, nr, dc, seg, cr, inv, fl, a8, nb, cra))

    def blk_copy(t, k, buf, sem):
        b, _, _, dc, _, _, _, _, a8, _, _ = tasks[t]
        sblk = a8 + k * BLKR
        sc = pl.multiple_of(jnp.minimum(sblk, S - BLKR), 8)
        return pltpu.make_async_copy(
            x_hbm.at[b, pl.ds(sc, BLKR), pl.ds(dc, DC)], buf, sem)

    def cls_copy(t):
        b, _, _, dc, _, _, _, _, _, _, cra = tasks[t]
        return pltpu.make_async_copy(
            x_hbm.at[b, pl.ds(cra, 8), pl.ds(dc, DC)], clsbuf, semC)

    def prime(t):
        @pl.when(tasks[t][9] > 0)
        def _():
            blk_copy(t, 0, bufA, semA).start()

    prime(0)
    cls_copy(0).start()
    prev_out_cp = None

    for slot in range(SLOTS):
        b, s0, nr, dc, seg, cr, inv, fl, a8, nb, cra = tasks[slot]

        def accum(k, acc, buf, s0=s0, nr=nr, a8=a8):
            sblk = a8 + k * BLKR
            sc = jnp.minimum(sblk, S - BLKR)
            lo = jnp.maximum(s0, sblk) - sc
            hi = jnp.minimum(s0 + nr, sblk + BLKR) - sc

            def rbody(r, a):
                return tuple(a[c] + buf[r, pl.ds(c * NLANE, NLANE)]
                             for c in range(NVEC))

            return lax.fori_loop(lo, hi, rbody, acc)

        def bbody(j, acc, slot=slot, nb=nb):
            k0 = 2 * j

            @pl.when(k0 + 1 < nb)
            def _():
                blk_copy(slot, k0 + 1, bufB, semB).start()

            blk_copy(slot, k0, bufA, semA).wait()
            acc = accum(k0, acc, bufA)

            @pl.when(k0 + 2 < nb)
            def _():
                blk_copy(slot, k0 + 2, bufA, semA).start()

            @pl.when(k0 + 1 < nb)
            def _():
                blk_copy(slot, k0 + 1, bufB, semB).wait()

            return accum(k0 + 1, acc, bufB)

        acc0 = tuple(jnp.zeros((NLANE,), jnp.float32) for _ in range(NVEC))
        nj = lax.shift_right_logical(nb + 1, 1)
        acc = lax.fori_loop(0, nj, bbody, acc0)

        if slot + 1 < SLOTS:
            prime(slot + 1)

        cls_copy(slot).wait()
        if prev_out_cp is not None:
            prev_out_cp.wait()
        coff = cr - cra
        for c in range(NVEC):
            outv[pl.ds(c * NLANE, NLANE)] = (
                acc[c] + fl * clsbuf[coff, pl.ds(c * NLANE, NLANE)]) * inv
        prev_out_cp = pltpu.make_async_copy(
            outv, out_hbm.at[pl.ds(pl.multiple_of(seg * D + dc, DC), DC)],
            semD)
        prev_out_cp.start()
        if slot + 1 < SLOTS:
            cls_copy(slot + 1).start()

    prev_out_cp.wait()


def _build_meta(h0, h1, r0, r1, t0, t1):
    barr = jnp.arange(B, dtype=jnp.int32)
    s0f = jnp.concatenate([h0, r0, t0])
    s1 = jnp.concatenate([h1, r1, t1])
    wf = s1 - s0f + 1
    s0 = jnp.maximum(s0f, S_TC)
    w = jnp.maximum(s1 - s0 + 1, 0)
    flag = jnp.concatenate([jnp.zeros(B), jnp.ones(B), jnp.ones(B)]
                           ).astype(jnp.float32)
    clsrow = jnp.concatenate([h0, h0, h0])
    flag = flag * (clsrow >= S_TC).astype(jnp.float32)
    segb = jnp.concatenate([barr, barr, barr])
    inv = 1.0 / (wf.astype(jnp.float32) +
                 jnp.concatenate([jnp.zeros(B), jnp.ones(B), jnp.ones(B)]))
    segid = jnp.arange(NSEG, dtype=jnp.int32)

    tseg = jnp.repeat(segid, NCH)
    tdc = jnp.tile(jnp.arange(NCH, dtype=jnp.int32) * DC, NSEG)
    tb = jnp.repeat(segb, NCH)
    ts0 = jnp.repeat(s0, NCH)
    tw = jnp.repeat(w, NCH)
    tcls = jnp.repeat(clsrow, NCH)
    tflag = jnp.repeat(flag, NCH)
    tinv = jnp.repeat(inv, NCH)

    fi = jnp.stack([tb, ts0, tw, tdc, tseg, tcls,
                    jnp.zeros_like(tb), jnp.zeros_like(tb)],
                   axis=-1).astype(jnp.int32)
    ff = jnp.stack([tinv, tflag], axis=-1).astype(jnp.float32)

    perm = jnp.argsort(-tw)
    cols = jnp.arange(NTILE).reshape(2, NTILE // 2).T.ravel()
    serp = jnp.stack([cols if r % 2 == 0 else cols[::-1]
                      for r in range(SLOTS)])
    pos = jnp.argsort(serp, axis=1) + (jnp.arange(SLOTS)[:, None] * NTILE)
    assign = perm[pos].T

    mi = jnp.broadcast_to(fi[assign][..., None],
                          (NTILE, SLOTS, 8, NLANE)).astype(jnp.int32)
    mf = jnp.broadcast_to(ff[assign][..., None],
                          (NTILE, SLOTS, 2, NLANE)).astype(jnp.float32)
    return mi, mf


@functools.cache
def _pool_kernel():
    return functools.partial(
        pl.kernel,
        out_type=jax.ShapeDtypeStruct((NSEG * D,), jnp.float32),
        mesh=plsc.VectorSubcoreMesh(core_axis_name="c", subcore_axis_name="s"),
        scratch_types=[
            pltpu.VMEM((SLOTS, 8, NLANE), jnp.int32),
            pltpu.VMEM((SLOTS, 2, NLANE), jnp.float32),
            pltpu.VMEM((BLKR, DC), jnp.float32),
            pltpu.VMEM((BLKR, DC), jnp.float32),
            pltpu.VMEM((8, DC), jnp.float32),
            pltpu.VMEM((DC,), jnp.float32),
            pltpu.SemaphoreType.DMA,
            pltpu.SemaphoreType.DMA,
            pltpu.SemaphoreType.DMA,
            pltpu.SemaphoreType.DMA,
        ],
    )(_sc_body)


def _tc_body(spans_ref, blk_ref, valid_ref, x_ref, out_ref):
    s = pl.program_id(1)
    b = pl.program_id(0)
    h0 = spans_ref[b, 0]
    h1 = spans_ref[b, 1]
    r0 = spans_ref[b, 2]
    r1 = spans_ref[b, 3]
    t0 = spans_ref[b, 4]
    t1 = spans_ref[b, 5]

    sblk = blk_ref[b, s]
    validf = valid_ref[b, s].astype(jnp.float32)
    pos = jax.lax.broadcasted_iota(jnp.int32, (1, BS_TC), 1) + sblk * BS_TC
    onehot = (pos == h0).astype(jnp.float32)

    def span_w(lo, hi, with_cls):
        m = ((pos >= lo) & (pos <= hi)).astype(jnp.float32)
        return m + onehot if with_cls else m

    wts = validf * jnp.concatenate([
        span_w(h0, h1, False),
        span_w(r0, r1, True),
        span_w(t0, t1, True),
    ], axis=0)

    contrib = jnp.dot(wts, x_ref[0], preferred_element_type=jnp.float32)

    @pl.when(s == 0)
    def _():
        out_ref[...] = jnp.zeros_like(out_ref)

    out_ref[0] += contrib

    @pl.when(s == (S_TC // BS_TC) - 1)
    def _():
        inv_h = 1.0 / (h1 - h0 + 1).astype(jnp.float32)
        inv_r = 1.0 / (r1 - r0 + 2).astype(jnp.float32)
        inv_t = 1.0 / (t1 - t0 + 2).astype(jnp.float32)
        out_ref[0, 0:1, :] *= inv_h
        out_ref[0, 1:2, :] *= inv_r
        out_ref[0, 2:3, :] *= inv_t


def _build_blockmap(spans):
    nblk = S_TC // BS_TC
    blk = jnp.arange(nblk, dtype=jnp.int32)
    lo = spans[:, 0::2].astype(jnp.int32)
    hi = jnp.minimum(spans[:, 1::2], S_TC - 1).astype(jnp.int32)
    touched = ((blk[None, :, None] * BS_TC <= hi[:, None, :]) &
               ((blk[None, :, None] + 1) * BS_TC > lo[:, None, :]) &
               (lo <= hi)[:, None, :]).any(-1)
    key = jnp.where(touched, blk[None, :], nblk)
    srt = jnp.sort(key, axis=1)
    cnt = touched.sum(axis=1, keepdims=True)
    pos = jnp.minimum(blk[None, :], jnp.maximum(cnt - 1, 0))
    blkmap = jnp.take_along_axis(srt, pos, axis=1)
    blkmap = jnp.where(blkmap == nblk, 0, blkmap).astype(jnp.int32)
    valid = (blk[None, :] < cnt).astype(jnp.int32)
    return blkmap, valid


@functools.cache
def _tc_kernel():
    return pl.pallas_call(
        _tc_body,
        grid_spec=pltpu.PrefetchScalarGridSpec(
            num_scalar_prefetch=3,
            grid=(B, S_TC // BS_TC),
            in_specs=[pl.BlockSpec((1, BS_TC, D),
                                   lambda b, s, spans, blk, valid:
                                   (b, blk[b, s], 0))],
            out_specs=pl.BlockSpec((1, 3, D),
                                   lambda b, s, *_: (b, 0, 0)),
        ),
        out_shape=jax.ShapeDtypeStruct((B, 3, D), jnp.float32),
        compiler_params=pltpu.CompilerParams(
            dimension_semantics=("parallel", "arbitrary")),
    )


def kernel(last_hidden_state, head_text_idx, relation_text_idx, tail_text_idx):
    hidx = head_text_idx.astype(jnp.int32)
    ridx = relation_text_idx.astype(jnp.int32)
    tidx = tail_text_idx.astype(jnp.int32)
    mi, mf = _build_meta(hidx[:, 0], hidx[:, 1], ridx[:, 0], ridx[:, 1],
                         tidx[:, 0], tidx[:, 1])
    sc_out = _pool_kernel()(last_hidden_state, mi, mf).reshape(NSEG, D)

    spans = jnp.concatenate([hidx, ridx, tidx], axis=-1)
    spans = jnp.pad(spans, ((0, 0), (0, 2)))
    blkmap, valid = _build_blockmap(spans)
    tc_out = _tc_kernel()(spans, blkmap, valid, last_hidden_state)

    head = tc_out[:, 0, :] + sc_out[:B]
    rel = tc_out[:, 1, :] + sc_out[B:2 * B]
    tail = tc_out[:, 2, :] + sc_out[2 * B:]
    return (head, rel, tail)


# speedup vs baseline: 1.0613x; 1.0148x over previous
import functools

import jax
import jax.numpy as jnp
from jax import lax
from jax.experimental import pallas as pl
from jax.experimental.pallas import tpu as pltpu
from jax.experimental.pallas import tpu_sc as plsc

B = 16
S = 4096
D = 1024
S_TC = 2048
DC = 256
NCH = D // DC
NSEG = 3 * B
NTASK = NSEG * NCH
NTILE = 32
SLOTS = NTASK // NTILE
BLKR = 128
LOG2_BLKR = 7
NLANE = 16
NVEC = DC // NLANE
BS_TC = 1024


def _sc_body(x_hbm, mi_hbm, mf_hbm, out_hbm, miv, mfv, bufA, bufB, clsbuf,
             outv, semA, semB, semC, semD):
    wid = lax.axis_index("c") * 16 + lax.axis_index("s")
    pltpu.sync_copy(mi_hbm.at[wid], miv)
    pltpu.sync_copy(mf_hbm.at[wid], mfv)

    tasks = []
    for slot in range(SLOTS):
        vi = miv[slot, :]
        vf = mfv[slot, :]
        b = vi[0]
        s0 = vi[1]
        nr = vi[2]
        dc = pl.multiple_of(vi[3], DC)
        seg = vi[4]
        cr = vi[5]
        inv = vf[0]
        fl = vf[1]
        a8 = jnp.bitwise_and(s0, jnp.int32(~7))
        nb = lax.shift_right_logical(s0 - a8 + nr + (BLKR - 1), LOG2_BLKR)
        nb = jnp.where(nr > 0, nb, 0)
        cra = pl.multiple_of(jnp.bitwise_and(cr, jnp.int32(~7)), 8)
        tasks.append((b, s0, nr, dc, seg, cr, inv, fl, a8, nb, cra))

    def blk_copy(t, k, buf, sem):
        b, _, _, dc, _, _, _, _, a8, _, _ = tasks[t]
        sblk = a8 + k * BLKR
        sc = pl.multiple_of(jnp.minimum(sblk, S - BLKR), 8)
        return pltpu.make_async_copy(
            x_hbm.at[b, pl.ds(sc, BLKR), pl.ds(dc, DC)], buf, sem)

    def cls_copy(t):
        b, _, _, dc, _, _, _, _, _, _, cra = tasks[t]
        return pltpu.make_async_copy(
            x_hbm.at[b, pl.ds(cra, 8), pl.ds(dc, DC)], clsbuf, semC)

    def prime(t):
        @pl.when(tasks[t][9] > 0)
        def _():
            blk_copy(t, 0, bufA, semA).start()

    prime(0)
    cls_copy(0).start()
    prev_out_cp = None

    for slot in range(SLOTS):
        b, s0, nr, dc, seg, cr, inv, fl, a8, nb, cra = tasks[slot]

        def accum(k, acc, buf, s0=s0, nr=nr, a8=a8):
            sblk = a8 + k * BLKR
            sc = jnp.minimum(sblk, S - BLKR)
            lo = jnp.maximum(s0, sblk) - sc
            hi = jnp.minimum(s0 + nr, sblk + BLKR) - sc

            def rbody(r, a):
                return tuple(a[c] + buf[r, pl.ds(c * NLANE, NLANE)]
                             for c in range(NVEC))

            return lax.fori_loop(lo, hi, rbody, acc)

        def bbody(j, acc, slot=slot, nb=nb):
            k0 = 2 * j

            @pl.when(k0 + 1 < nb)
            def _():
                blk_copy(slot, k0 + 1, bufB, semB).start()

            blk_copy(slot, k0, bufA, semA).wait()
            acc = accum(k0, acc, bufA)

            @pl.when(k0 + 2 < nb)
            def _():
                blk_copy(slot, k0 + 2, bufA, semA).start()

            @pl.when(k0 + 1 < nb)
            def _():
                blk_copy(slot, k0 + 1, bufB, semB).wait()

            return accum(k0 + 1, acc, bufB)

        acc0 = tuple(jnp.zeros((NLANE,), jnp.float32) for _ in range(NVEC))
        nj = lax.shift_right_logical(nb + 1, 1)
        acc = lax.fori_loop(0, nj, bbody, acc0)

        if slot + 1 < SLOTS:
            prime(slot + 1)

        cls_copy(slot).wait()
        if prev_out_cp is not None:
            prev_out_cp.wait()
        coff = cr - cra
        for c in range(NVEC):
            outv[pl.ds(c * NLANE, NLANE)] = (
                acc[c] + fl * clsbuf[coff, pl.ds(c * NLANE, NLANE)]) * inv
        prev_out_cp = pltpu.make_async_copy(
            outv, out_hbm.at[pl.ds(pl.multiple_of(seg * D + dc, DC), DC)],
            semD)
        prev_out_cp.start()
        if slot + 1 < SLOTS:
            cls_copy(slot + 1).start()

    prev_out_cp.wait()


def _build_meta(h0, h1, r0, r1, t0, t1):
    barr = jnp.arange(B, dtype=jnp.int32)
    s0f = jnp.concatenate([h0, r0, t0])
    s1 = jnp.concatenate([h1, r1, t1])
    wf = s1 - s0f + 1
    s0 = jnp.maximum(s0f, S_TC)
    w = jnp.maximum(s1 - s0 + 1, 0)
    flag = jnp.concatenate([jnp.zeros(B), jnp.ones(B), jnp.ones(B)]
                           ).astype(jnp.float32)
    clsrow = jnp.concatenate([h0, h0, h0])
    flag = flag * (clsrow >= S_TC).astype(jnp.float32)
    segb = jnp.concatenate([barr, barr, barr])
    inv = 1.0 / (wf.astype(jnp.float32) +
                 jnp.concatenate([jnp.zeros(B), jnp.ones(B), jnp.ones(B)]))
    segid = jnp.arange(NSEG, dtype=jnp.int32)

    tseg = jnp.repeat(segid, NCH)
    tdc = jnp.tile(jnp.arange(NCH, dtype=jnp.int32) * DC, NSEG)
    tb = jnp.repeat(segb, NCH)
    ts0 = jnp.repeat(s0, NCH)
    tw = jnp.repeat(w, NCH)
    tcls = jnp.repeat(clsrow, NCH)
    tflag = jnp.repeat(flag, NCH)
    tinv = jnp.repeat(inv, NCH)

    zi = jnp.zeros_like(tb)
    fi = jnp.stack([tb, ts0, tw, tdc, tseg, tcls, zi, zi, zi, zi, zi, zi,
                    zi, zi, zi, zi], axis=-1).astype(jnp.int32)
    zf = jnp.zeros_like(tinv)
    ff = jnp.stack([tinv, tflag] + [zf] * 14,
                   axis=-1).astype(jnp.float32)

    perm = jnp.argsort(-tw)
    cols = jnp.arange(NTILE).reshape(2, NTILE // 2).T.ravel()
    serp = jnp.stack([cols if r % 2 == 0 else cols[::-1]
                      for r in range(SLOTS)])
    pos = jnp.argsort(serp, axis=1) + (jnp.arange(SLOTS)[:, None] * NTILE)
    assign = perm[pos].T

    mi = fi[assign]
    mf = ff[assign]
    return mi, mf


@functools.cache
def _pool_kernel():
    return functools.partial(
        pl.kernel,
        out_type=jax.ShapeDtypeStruct((NSEG * D,), jnp.float32),
        mesh=plsc.VectorSubcoreMesh(core_axis_name="c", subcore_axis_name="s"),
        scratch_types=[
            pltpu.VMEM((SLOTS, NLANE), jnp.int32),
            pltpu.VMEM((SLOTS, NLANE), jnp.float32),
            pltpu.VMEM((BLKR, DC), jnp.float32),
            pltpu.VMEM((BLKR, DC), jnp.float32),
            pltpu.VMEM((8, DC), jnp.float32),
            pltpu.VMEM((DC,), jnp.float32),
            pltpu.SemaphoreType.DMA,
            pltpu.SemaphoreType.DMA,
            pltpu.SemaphoreType.DMA,
            pltpu.SemaphoreType.DMA,
        ],
    )(_sc_body)


def _tc_body(spans_ref, blk_ref, valid_ref, x_ref, out_ref):
    s = pl.program_id(1)
    b = pl.program_id(0)
    h0 = spans_ref[b, 0]
    h1 = spans_ref[b, 1]
    r0 = spans_ref[b, 2]
    r1 = spans_ref[b, 3]
    t0 = spans_ref[b, 4]
    t1 = spans_ref[b, 5]

    sblk = blk_ref[b, s]
    validf = valid_ref[b, s].astype(jnp.float32)
    pos = jax.lax.broadcasted_iota(jnp.int32, (1, BS_TC), 1) + sblk * BS_TC
    onehot = (pos == h0).astype(jnp.float32)

    def span_w(lo, hi, with_cls):
        m = ((pos >= lo) & (pos <= hi)).astype(jnp.float32)
        return m + onehot if with_cls else m

    wts = validf * jnp.concatenate([
        span_w(h0, h1, False),
        span_w(r0, r1, True),
        span_w(t0, t1, True),
    ], axis=0)

    contrib = jnp.dot(wts, x_ref[0], preferred_element_type=jnp.float32)

    @pl.when(s == 0)
    def _():
        out_ref[...] = jnp.zeros_like(out_ref)

    out_ref[0] += contrib

    @pl.when(s == (S_TC // BS_TC) - 1)
    def _():
        inv_h = 1.0 / (h1 - h0 + 1).astype(jnp.float32)
        inv_r = 1.0 / (r1 - r0 + 2).astype(jnp.float32)
        inv_t = 1.0 / (t1 - t0 + 2).astype(jnp.float32)
        out_ref[0, 0:1, :] *= inv_h
        out_ref[0, 1:2, :] *= inv_r
        out_ref[0, 2:3, :] *= inv_t


def _build_blockmap(spans):
    nblk = S_TC // BS_TC
    blk = jnp.arange(nblk, dtype=jnp.int32)
    lo = spans[:, 0::2].astype(jnp.int32)
    hi = jnp.minimum(spans[:, 1::2], S_TC - 1).astype(jnp.int32)
    touched = ((blk[None, :, None] * BS_TC <= hi[:, None, :]) &
               ((blk[None, :, None] + 1) * BS_TC > lo[:, None, :]) &
               (lo <= hi)[:, None, :]).any(-1)
    key = jnp.where(touched, blk[None, :], nblk)
    srt = jnp.sort(key, axis=1)
    cnt = touched.sum(axis=1, keepdims=True)
    pos = jnp.minimum(blk[None, :], jnp.maximum(cnt - 1, 0))
    blkmap = jnp.take_along_axis(srt, pos, axis=1)
    blkmap = jnp.where(blkmap == nblk, 0, blkmap).astype(jnp.int32)
    valid = (blk[None, :] < cnt).astype(jnp.int32)
    return blkmap, valid


@functools.cache
def _tc_kernel():
    return pl.pallas_call(
        _tc_body,
        grid_spec=pltpu.PrefetchScalarGridSpec(
            num_scalar_prefetch=3,
            grid=(B, S_TC // BS_TC),
            in_specs=[pl.BlockSpec((1, BS_TC, D),
                                   lambda b, s, spans, blk, valid:
                                   (b, blk[b, s], 0))],
            out_specs=pl.BlockSpec((1, 3, D),
                                   lambda b, s, *_: (b, 0, 0)),
        ),
        out_shape=jax.ShapeDtypeStruct((B, 3, D), jnp.float32),
        compiler_params=pltpu.CompilerParams(
            dimension_semantics=("parallel", "arbitrary")),
    )


def kernel(last_hidden_state, head_text_idx, relation_text_idx, tail_text_idx):
    hidx = head_text_idx.astype(jnp.int32)
    ridx = relation_text_idx.astype(jnp.int32)
    tidx = tail_text_idx.astype(jnp.int32)
    mi, mf = _build_meta(hidx[:, 0], hidx[:, 1], ridx[:, 0], ridx[:, 1],
                         tidx[:, 0], tidx[:, 1])
    sc_out = _pool_kernel()(last_hidden_state, mi, mf).reshape(NSEG, D)

    spans = jnp.concatenate([hidx, ridx, tidx], axis=-1)
    spans = jnp.pad(spans, ((0, 0), (0, 2)))
    blkmap, valid = _build_blockmap(spans)
    tc_out = _tc_kernel()(spans, blkmap, valid, last_hidden_state)

    head = tc_out[:, 0, :] + sc_out[:B]
    rel = tc_out[:, 1, :] + sc_out[B:2 * B]
    tail = tc_out[:, 2, :] + sc_out[2 * B:]
    return (head, rel, tail)
